# baseline (speedup 1.0000x reference)
import jax
import jax.numpy as jnp
from jax.experimental import pallas as pl

N_TOKENS = 32768
D_MODEL = 768
N_EXP = 8
BLK = 2048


def _router_blk(x_ref, w_ref, logits_ref, probs_ref, ent_ref):
    x = x_ref[...]
    w = w_ref[...]
    logits_t = jax.lax.dot_general(
        w, x, (((1,), (1,)), ((), ())),
        preferred_element_type=jnp.float32)
    m = jnp.max(logits_t, axis=0, keepdims=True)
    e = jnp.exp(logits_t - m)
    s = jnp.sum(e, axis=0, keepdims=True)
    probs_t = e * (1.0 / s)
    logits_ref[...] = logits_t.T
    probs_ref[...] = probs_t.T
    plsum = jnp.sum(probs_t * logits_t, axis=0, keepdims=True)
    ent_ref[0, ...] = m + jnp.log(s) - plsum


def kernel(x, W):
    grid = N_TOKENS // BLK
    logits, probs, ent_parts = pl.pallas_call(
        _router_blk,
        grid=(grid,),
        in_specs=[
            pl.BlockSpec((BLK, D_MODEL), lambda i: (i, 0)),
            pl.BlockSpec((N_EXP, D_MODEL), lambda i: (0, 0)),
        ],
        out_specs=[
            pl.BlockSpec((BLK, N_EXP), lambda i: (i, 0)),
            pl.BlockSpec((BLK, N_EXP), lambda i: (i, 0)),
            pl.BlockSpec((1, 1, BLK), lambda i: (i, 0, 0)),
        ],
        out_shape=[
            jax.ShapeDtypeStruct((N_TOKENS, N_EXP), jnp.float32),
            jax.ShapeDtypeStruct((N_TOKENS, N_EXP), jnp.float32),
            jax.ShapeDtypeStruct((grid, 1, BLK), jnp.float32),
        ],
    )(x, W)
    router_entropy = jnp.sum(ent_parts) / N_TOKENS
    return (logits, probs, router_entropy)


# speedup vs baseline: 1.8391x; 1.8391x over previous
import jax
import jax.numpy as jnp
from jax.experimental import pallas as pl

N_TOKENS = 32768
D_MODEL = 768
N_EXP = 8
BLK = 4096


def _router_blk(x_ref, w_ref, logits_ref, probs_ref, ent_ref):
    x = x_ref[...]
    w = w_ref[...]
    logits_t = jax.lax.dot_general(
        w, x, (((1,), (1,)), ((), ())),
        preferred_element_type=jnp.float32)
    m = jnp.max(logits_t, axis=0, keepdims=True)
    e = jnp.exp(logits_t - m)
    s = jnp.sum(e, axis=0, keepdims=True)
    probs_t = e * (1.0 / s)
    logits_ref[...] = logits_t
    probs_ref[...] = probs_t
    plsum = jnp.sum(probs_t * logits_t, axis=0, keepdims=True)
    ent_ref[0, ...] = m + jnp.log(s) - plsum


def kernel(x, W):
    grid = N_TOKENS // BLK
    logits_t, probs_t, ent_parts = pl.pallas_call(
        _router_blk,
        grid=(grid,),
        in_specs=[
            pl.BlockSpec((BLK, D_MODEL), lambda i: (i, 0)),
            pl.BlockSpec((N_EXP, D_MODEL), lambda i: (0, 0)),
        ],
        out_specs=[
            pl.BlockSpec((N_EXP, BLK), lambda i: (0, i)),
            pl.BlockSpec((N_EXP, BLK), lambda i: (0, i)),
            pl.BlockSpec((1, 1, BLK), lambda i: (i, 0, 0)),
        ],
        out_shape=[
            jax.ShapeDtypeStruct((N_EXP, N_TOKENS), jnp.float32),
            jax.ShapeDtypeStruct((N_EXP, N_TOKENS), jnp.float32),
            jax.ShapeDtypeStruct((grid, 1, BLK), jnp.float32),
        ],
    )(x, W)
    router_entropy = jnp.sum(ent_parts) / N_TOKENS
    return (logits_t.T, probs_t.T, router_entropy)
